# Initial kernel scaffold; baseline (speedup 1.0000x reference)
import jax
import jax.numpy as jnp
from jax.experimental import pallas as pl


def kernel(x, idx, x1):
    raise NotImplementedError("write your pallas kernel here")


# speedup vs baseline: 37.9419x; 37.9419x over previous
import functools

import jax
import jax.numpy as jnp
from jax import lax
from jax.experimental import pallas as pl
from jax.experimental.pallas import tpu as pltpu
from jax.experimental.pallas import tpu_sc as plsc

NC = 2
NS = 16
L = 16
NW = NC * NS


@functools.partial(jax.jit, static_argnums=(2, 3))
def _unpool(x_flat, idx_flat, n_out, n_quarters):
    P, N_IN = x_flat.shape
    QUARTER = n_out // n_quarters
    TASKS = P * n_quarters
    CH = 8192

    mesh = plsc.VectorSubcoreMesh(core_axis_name="c", subcore_axis_name="s")

    @functools.partial(
        pl.kernel,
        out_type=jax.ShapeDtypeStruct((P, n_out), jnp.float32),
        mesh=mesh,
        compiler_params=pltpu.CompilerParams(needs_layout_passes=False),
        scratch_types=[
            pltpu.VMEM((QUARTER,), jnp.float32),
            pltpu.VMEM((CH,), jnp.int32),
            pltpu.VMEM((CH,), jnp.float32),
        ],
    )
    def body(x_hbm, idx_hbm, out_hbm, qbuf, ibuf, xbuf):
        wid = lax.axis_index("s") * NC + lax.axis_index("c")
        zeros = jnp.zeros((L,), jnp.float32)

        def task_loop(t, carry):
            task = wid + t * NW
            p = task // n_quarters
            q = task % n_quarters
            base = q * QUARTER

            def zloop(i, c):
                qbuf[pl.ds(i * (4 * L), L)] = zeros
                qbuf[pl.ds(i * (4 * L) + L, L)] = zeros
                qbuf[pl.ds(i * (4 * L) + 2 * L, L)] = zeros
                qbuf[pl.ds(i * (4 * L) + 3 * L, L)] = zeros
                return c

            lax.fori_loop(0, QUARTER // (4 * L), zloop, 0)

            def chunk_loop(ci, c):
                off = ci * CH
                pltpu.sync_copy(idx_hbm.at[p, pl.ds(off, CH)], ibuf)
                pltpu.sync_copy(x_hbm.at[p, pl.ds(off, CH)], xbuf)

                def scan_loop(j, cc):
                    for k in range(4):
                        o = j * (4 * L) + k * L
                        iv = ibuf[pl.ds(o, L)]
                        xv = xbuf[pl.ds(o, L)]
                        local = iv - base
                        m = (local >= 0) & (local < QUARTER)
                        localc = jnp.minimum(
                            jnp.maximum(local, 0), QUARTER - 1
                        )
                        plsc.store_scatter(qbuf, [localc], xv, mask=m)
                    return cc

                lax.fori_loop(0, CH // (4 * L), scan_loop, 0)
                return c

            lax.fori_loop(0, N_IN // CH, chunk_loop, 0)
            pltpu.sync_copy(qbuf, out_hbm.at[p, pl.ds(base, QUARTER)])
            return carry

        lax.fori_loop(0, TASKS // NW, task_loop, 0)

    return body(x_flat, idx_flat)


def kernel(x, idx, x1):
    B, C, H, W = x.shape
    Hout, Wout = x1.shape[2], x1.shape[3]
    n_out = Hout * Wout
    xf = x.reshape(B * C, H * W)
    idf = idx.reshape(B * C, H * W).astype(jnp.int32)
    out = _unpool(xf, idf, n_out, 4)
    return out.reshape(B, C, Hout, Wout)
